# baseline (speedup 1.0000x reference)
import functools

import jax
import jax.numpy as jnp
from jax.experimental import pallas as pl
from jax.experimental.pallas import tpu as pltpu


def _sage_kernel(adj_ref, xb_ref, w1_ref, w2_ref, wm_ref, b_ref,
                 out_ref, *, m_blk):
    i = pl.program_id(0)
    a = adj_ref[...]
    agg = jnp.dot(a, xb_ref[...],
                  precision=jax.lax.Precision.DEFAULT,
                  preferred_element_type=jnp.float32)
    f = w1_ref.shape[0]
    neigh = agg[:, :f] / (agg[:, f:f + 1] + 1.0)
    xi = xb_ref[pl.ds(i * m_blk, m_blk), :f]
    h = (jnp.dot(xi, w1_ref[...], preferred_element_type=jnp.float32)
         + jnp.dot(neigh, w2_ref[...], preferred_element_type=jnp.float32))
    h = jnp.maximum(h, 0.0)
    out_ref[...] = (jnp.dot(h, wm_ref[...], preferred_element_type=jnp.float32)
                    + b_ref[...])


@jax.jit
def kernel(x, adj, W_sage, W_mlp, b_mlp):
    n, f = x.shape
    h_dim = W_sage.shape[0]
    c = W_mlp.shape[0]

    m_blk = 400 if n % 400 == 0 else n

    w1t = W_sage[:, :f].T
    w2t = W_sage[:, f:].T
    wmt = W_mlp.T
    b = b_mlp.reshape(1, c)

    out = pl.pallas_call(
        functools.partial(_sage_kernel, m_blk=m_blk),
        grid=(n // m_blk,),
        in_specs=[
            pl.BlockSpec((m_blk, n), lambda i: (i, 0)),
            pl.BlockSpec((n, f + 1), lambda i: (0, 0)),
            pl.BlockSpec((f, h_dim), lambda i: (0, 0)),
            pl.BlockSpec((f, h_dim), lambda i: (0, 0)),
            pl.BlockSpec((h_dim, c), lambda i: (0, 0)),
            pl.BlockSpec((1, c), lambda i: (0, 0)),
        ],
        out_specs=pl.BlockSpec((m_blk, c), lambda i: (i, 0)),
        out_shape=jax.ShapeDtypeStruct((n, c), jnp.float32),
        compiler_params=pltpu.CompilerParams(
            dimension_semantics=("parallel",)),
    )(adj, jnp.concatenate([x, jnp.ones((n, 1), x.dtype)], axis=1),
      w1t, w2t, wmt, b)
    return out


# speedup vs baseline: 1.0841x; 1.0841x over previous
import functools

import jax
import jax.numpy as jnp
from jax.experimental import pallas as pl
from jax.experimental.pallas import tpu as pltpu


def _sage_kernel(adj_ref, xb_ref, w1_ref, w2_ref, wm_ref, b_ref,
                 out_ref, *, m_blk):
    i = pl.program_id(0)
    a = adj_ref[...]
    deg = jnp.sum(a, axis=1, keepdims=True)
    neigh = jnp.dot(a, xb_ref[...],
                    precision=jax.lax.Precision.DEFAULT,
                    preferred_element_type=jnp.float32)
    neigh = neigh / (deg + 1.0)
    xi = xb_ref[pl.ds(i * m_blk, m_blk), :]
    h = (jnp.dot(xi, w1_ref[...], preferred_element_type=jnp.float32)
         + jnp.dot(neigh, w2_ref[...], preferred_element_type=jnp.float32))
    h = jnp.maximum(h, 0.0)
    out_ref[...] = (jnp.dot(h, wm_ref[...], preferred_element_type=jnp.float32)
                    + b_ref[...])


@jax.jit
def kernel(x, adj, W_sage, W_mlp, b_mlp):
    n, f = x.shape
    h_dim = W_sage.shape[0]
    c = W_mlp.shape[0]

    m_blk = 200 if n % 200 == 0 else n

    w1t = W_sage[:, :f].T
    w2t = W_sage[:, f:].T
    wmt = W_mlp.T
    b = b_mlp.reshape(1, c)

    out = pl.pallas_call(
        functools.partial(_sage_kernel, m_blk=m_blk),
        grid=(n // m_blk,),
        in_specs=[
            pl.BlockSpec((m_blk, n), lambda i: (i, 0)),
            pl.BlockSpec((n, f), lambda i: (0, 0)),
            pl.BlockSpec((f, h_dim), lambda i: (0, 0)),
            pl.BlockSpec((f, h_dim), lambda i: (0, 0)),
            pl.BlockSpec((h_dim, c), lambda i: (0, 0)),
            pl.BlockSpec((1, c), lambda i: (0, 0)),
        ],
        out_specs=pl.BlockSpec((m_blk, c), lambda i: (i, 0)),
        out_shape=jax.ShapeDtypeStruct((n, c), jnp.float32),
        compiler_params=pltpu.CompilerParams(
            dimension_semantics=("parallel",)),
    )(adj, x, w1t, w2t, wmt, b)
    return out


# speedup vs baseline: 1.1814x; 1.0897x over previous
import functools

import jax
import jax.numpy as jnp
from jax.experimental import pallas as pl
from jax.experimental.pallas import tpu as pltpu


def _sage_kernel(adj_ref, xb_ref, w1_ref, w2_ref, wm_ref, b_ref,
                 out_ref, *, m_blk):
    i = pl.program_id(0)
    a = adj_ref[...]
    deg = jnp.sum(a, axis=1, keepdims=True)
    neigh = jnp.dot(a, xb_ref[...],
                    precision=jax.lax.Precision.DEFAULT,
                    preferred_element_type=jnp.float32)
    neigh = neigh / (deg + 1.0)
    xi = xb_ref[pl.ds(i * m_blk, m_blk), :]
    h = (jnp.dot(xi, w1_ref[...], preferred_element_type=jnp.float32)
         + jnp.dot(neigh, w2_ref[...], preferred_element_type=jnp.float32))
    h = jnp.maximum(h, 0.0)
    out_ref[...] = (jnp.dot(h, wm_ref[...], preferred_element_type=jnp.float32)
                    + b_ref[...])


@jax.jit
def kernel(x, adj, W_sage, W_mlp, b_mlp):
    n, f = x.shape
    h_dim = W_sage.shape[0]
    c = W_mlp.shape[0]

    m_blk = 400 if n % 400 == 0 else n

    w1t = W_sage[:, :f].T
    w2t = W_sage[:, f:].T
    wmt = W_mlp.T
    b = b_mlp.reshape(1, c)

    out = pl.pallas_call(
        functools.partial(_sage_kernel, m_blk=m_blk),
        grid=(n // m_blk,),
        in_specs=[
            pl.BlockSpec((m_blk, n), lambda i: (i, 0)),
            pl.BlockSpec((n, f), lambda i: (0, 0)),
            pl.BlockSpec((f, h_dim), lambda i: (0, 0)),
            pl.BlockSpec((f, h_dim), lambda i: (0, 0)),
            pl.BlockSpec((h_dim, c), lambda i: (0, 0)),
            pl.BlockSpec((1, c), lambda i: (0, 0)),
        ],
        out_specs=pl.BlockSpec((m_blk, c), lambda i: (i, 0)),
        out_shape=jax.ShapeDtypeStruct((n, c), jnp.float32),
        compiler_params=pltpu.CompilerParams(
            dimension_semantics=("parallel",)),
    )(adj, x, w1t, w2t, wmt, b)
    return out
